# Initial kernel scaffold; baseline (speedup 1.0000x reference)
import jax
import jax.numpy as jnp
from jax.experimental import pallas as pl


def kernel(x, edge_index, edge_attr, h, batch, W_l, b_l, W_r, b_r, gamma, beta):
    raise NotImplementedError("write your pallas kernel here")


# speedup vs baseline: 3.5018x; 3.5018x over previous
import functools

import jax
import jax.numpy as jnp
from jax import lax
from jax.experimental import pallas as pl
from jax.experimental.pallas import tpu as pltpu
from jax.experimental.pallas import tpu_sc as plsc

_N = 10000
_D = 128
_E = 320000

_NC = 2
_NS = 16
_NW = _NC * _NS
_CW = 144
_CHUNK = 128
_NP = 10240
_RPT = _NP // _NS
_CPW = 80
_EP = _NW * _CPW * _CHUNK
_BR = 512


def _tc_pre(x_ref, wcat_ref, g_ref, b_ref, bias_ref, zaug_ref, res_ref):
    xr = x_ref[...]
    mu = jnp.mean(xr, axis=1, keepdims=True)
    d = xr - mu
    var = jnp.mean(d * d, axis=1, keepdims=True)
    xn = d * lax.rsqrt(var + 1e-5) * g_ref[...] + b_ref[...]
    zz = lax.dot_general(xn, wcat_ref[...], (((1,), (0,)), ((), ())),
                         preferred_element_type=jnp.float32)
    res_ref[...] = zz[:, _D:] + xr + bias_ref[...]
    col = lax.broadcasted_iota(jnp.int32, (xr.shape[0], _CW - _D), 1)
    tail = jnp.where(col == 0, 1.0, 0.0).astype(jnp.float32)
    zaug_ref[:, :_D] = zz[:, :_D]
    zaug_ref[:, _D:] = tail


def _tc_post(acc_ref, res_ref, out_ref):
    s = acc_ref[0] + acc_ref[1]
    agg = s[:, :_D]
    deg = s[:, _D:_D + 1]
    mean = agg / jnp.maximum(deg, 1.0)
    out_ref[...] = jnp.maximum(mean + res_ref[...], 0.0)


def _sc_body(zaug_hbm, src_hbm, dst_hbm, zero_hbm, out_hbm,
             src_v, dst_v, rows_v, acc_sh, sem):
    c = lax.axis_index("c")
    s = lax.axis_index("s")
    wid = c * _NS + s
    pltpu.sync_copy(zero_hbm, acc_sh.at[pl.ds(s * _RPT, _RPT)])
    pltpu.sync_copy(src_hbm.at[wid], src_v)
    pltpu.sync_copy(dst_hbm.at[wid], dst_v)
    plsc.subcore_barrier()

    def body(j, carry):
        pltpu.async_copy(zaug_hbm.at[src_v.at[j]], rows_v, sem).wait()
        pltpu.sync_copy(rows_v, acc_sh.at[dst_v.at[j]], add=True)
        return carry

    lax.fori_loop(0, _CPW, body, 0)
    plsc.subcore_barrier()
    pltpu.sync_copy(acc_sh.at[pl.ds(s * _RPT, _RPT)],
                    out_hbm.at[c, pl.ds(s * _RPT, _RPT)])


@functools.cache
def _sc_scatter():
    return pl.kernel(
        _sc_body,
        out_type=jax.ShapeDtypeStruct((_NC, _NP, _CW), jnp.float32),
        mesh=plsc.VectorSubcoreMesh(core_axis_name="c", subcore_axis_name="s",
                                    num_cores=_NC, num_subcores=_NS),
        scratch_types=[
            pltpu.VMEM((_CPW, _CHUNK), jnp.int32),
            pltpu.VMEM((_CPW, _CHUNK), jnp.int32),
            pltpu.VMEM((_CHUNK, _CW), jnp.float32),
            pltpu.VMEM_SHARED((_NP, _CW), jnp.float32),
            pltpu.SemaphoreType.DMA,
        ],
        compiler_params=pltpu.CompilerParams(use_tc_tiling_on_sc=False),
    )


def kernel(x, edge_index, edge_attr, h, batch, W_l, b_l, W_r, b_r, gamma, beta):
    x_p = jnp.pad(x, ((0, _NP - _N), (0, 0)))
    wcat = jnp.concatenate([W_l.T, W_r.T], axis=1)
    bias = (b_l + b_r).reshape(1, _D)
    g = gamma.reshape(1, _D)
    b = beta.reshape(1, _D)
    grid = (_NP // _BR,)

    zaug, res = pl.pallas_call(
        _tc_pre,
        grid=grid,
        in_specs=[
            pl.BlockSpec((_BR, _D), lambda i: (i, 0)),
            pl.BlockSpec((_D, 2 * _D), lambda i: (0, 0)),
            pl.BlockSpec((1, _D), lambda i: (0, 0)),
            pl.BlockSpec((1, _D), lambda i: (0, 0)),
            pl.BlockSpec((1, _D), lambda i: (0, 0)),
        ],
        out_specs=[
            pl.BlockSpec((_BR, _CW), lambda i: (i, 0)),
            pl.BlockSpec((_BR, _D), lambda i: (i, 0)),
        ],
        out_shape=[
            jax.ShapeDtypeStruct((_NP, _CW), jnp.float32),
            jax.ShapeDtypeStruct((_NP, _D), jnp.float32),
        ],
    )(x_p, wcat, g, b, bias)

    pad = _EP - _E
    src3 = jnp.concatenate(
        [edge_index[0], jnp.zeros((pad,), jnp.int32)]).reshape(_NW, _CPW, _CHUNK)
    dst3 = jnp.concatenate(
        [edge_index[1], jnp.full((pad,), _N, jnp.int32)]).reshape(_NW, _CPW, _CHUNK)
    zero = jnp.zeros((_RPT, _CW), jnp.float32)

    acc = _sc_scatter()(zaug, src3, dst3, zero)

    out_p = pl.pallas_call(
        _tc_post,
        grid=grid,
        in_specs=[
            pl.BlockSpec((_NC, _BR, _CW), lambda i: (0, i, 0)),
            pl.BlockSpec((_BR, _D), lambda i: (i, 0)),
        ],
        out_specs=pl.BlockSpec((_BR, _D), lambda i: (i, 0)),
        out_shape=jax.ShapeDtypeStruct((_NP, _D), jnp.float32),
    )(acc, res)

    return (out_p[:_N], h)


# speedup vs baseline: 6.0370x; 1.7240x over previous
import functools

import jax
import jax.numpy as jnp
from jax import lax
from jax.experimental import pallas as pl
from jax.experimental.pallas import tpu as pltpu
from jax.experimental.pallas import tpu_sc as plsc

_N = 10000
_D = 128
_E = 320000

_NC = 2
_NS = 16
_NW = _NC * _NS
_CW = 144
_CHUNK = 64
_NBUF = 2
_NPT = 10240
_RPT = _N // _NS
_CPW = 158
_EP = _NW * _CPW * _CHUNK
_BR = 512
_BRP = 1000


def _tc_pre(x_ref, wcat_ref, g_ref, b_ref, bias_ref, zaug_ref, res_ref):
    xr = x_ref[...]
    mu = jnp.mean(xr, axis=1, keepdims=True)
    d = xr - mu
    var = jnp.mean(d * d, axis=1, keepdims=True)
    xn = d * lax.rsqrt(var + 1e-5) * g_ref[...] + b_ref[...]
    zz = lax.dot_general(xn, wcat_ref[...], (((1,), (0,)), ((), ())),
                         preferred_element_type=jnp.float32)
    res_ref[...] = zz[:, _D:] + xr + bias_ref[...]
    row = pl.program_id(0) * _BR + lax.broadcasted_iota(jnp.int32, (_BR, _D), 0)
    real = row < _N
    col = lax.broadcasted_iota(jnp.int32, (_BR, _CW - _D), 1)
    row_t = pl.program_id(0) * _BR + lax.broadcasted_iota(
        jnp.int32, (_BR, _CW - _D), 0)
    tail = jnp.where((col == 0) & (row_t < _N), 1.0, 0.0).astype(jnp.float32)
    zaug_ref[:, :_D] = jnp.where(real, zz[:, :_D], 0.0)
    zaug_ref[:, _D:] = tail


def _tc_post(acc_ref, res_ref, out_ref):
    s = acc_ref[0] + acc_ref[1]
    agg = s[:, :_D]
    deg = s[:, _D:_D + 1]
    mean = agg / jnp.maximum(deg, 1.0)
    out_ref[...] = jnp.maximum(mean + res_ref[...], 0.0)


def _sc_body(zaug_hbm, src_hbm, dst_hbm, zero_hbm, out_hbm,
             src_v, dst_v, rows_v, acc_sh, sems):
    c = lax.axis_index("c")
    s = lax.axis_index("s")
    wid = c * _NS + s
    pltpu.sync_copy(zero_hbm, acc_sh.at[pl.ds(s * _RPT, _RPT)])
    pltpu.sync_copy(src_hbm.at[wid], src_v)
    pltpu.sync_copy(dst_hbm.at[wid], dst_v)
    plsc.subcore_barrier()

    for b in range(_NBUF):
        pltpu.async_copy(zaug_hbm.at[src_v.at[b]], rows_v.at[b], sems.at[b])

    def body(t, carry):
        for b in range(_NBUF):
            j = t * _NBUF + b
            pltpu.make_async_copy(
                zaug_hbm.at[src_v.at[j]], rows_v.at[b], sems.at[b]).wait()
            pltpu.sync_copy(rows_v.at[b], acc_sh.at[dst_v.at[j]], add=True)

            @pl.when(j + _NBUF < _CPW)
            def _():
                pltpu.async_copy(
                    zaug_hbm.at[src_v.at[j + _NBUF]], rows_v.at[b], sems.at[b])
        return carry

    lax.fori_loop(0, _CPW // _NBUF, body, 0)
    plsc.subcore_barrier()
    pltpu.sync_copy(acc_sh.at[pl.ds(s * _RPT, _RPT)],
                    out_hbm.at[c, pl.ds(s * _RPT, _RPT)])


@functools.cache
def _sc_scatter():
    return pl.kernel(
        _sc_body,
        out_type=jax.ShapeDtypeStruct((_NC, _N, _CW), jnp.float32),
        mesh=plsc.VectorSubcoreMesh(core_axis_name="c", subcore_axis_name="s",
                                    num_cores=_NC, num_subcores=_NS),
        scratch_types=[
            pltpu.VMEM((_CPW, _CHUNK), jnp.int32),
            pltpu.VMEM((_CPW, _CHUNK), jnp.int32),
            pltpu.VMEM((_NBUF, _CHUNK, _CW), jnp.float32),
            pltpu.VMEM_SHARED((_N, _CW), jnp.float32),
            pltpu.SemaphoreType.DMA((_NBUF,)),
        ],
        compiler_params=pltpu.CompilerParams(use_tc_tiling_on_sc=False),
    )


def kernel(x, edge_index, edge_attr, h, batch, W_l, b_l, W_r, b_r, gamma, beta):
    x_p = jnp.pad(x, ((0, _NPT - _N), (0, 0)))
    wcat = jnp.concatenate([W_l.T, W_r.T], axis=1)
    bias = (b_l + b_r).reshape(1, _D)
    g = gamma.reshape(1, _D)
    b = beta.reshape(1, _D)
    grid = (_NPT // _BR,)

    zaug, res = pl.pallas_call(
        _tc_pre,
        grid=grid,
        in_specs=[
            pl.BlockSpec((_BR, _D), lambda i: (i, 0)),
            pl.BlockSpec((_D, 2 * _D), lambda i: (0, 0)),
            pl.BlockSpec((1, _D), lambda i: (0, 0)),
            pl.BlockSpec((1, _D), lambda i: (0, 0)),
            pl.BlockSpec((1, _D), lambda i: (0, 0)),
        ],
        out_specs=[
            pl.BlockSpec((_BR, _CW), lambda i: (i, 0)),
            pl.BlockSpec((_BR, _D), lambda i: (i, 0)),
        ],
        out_shape=[
            jax.ShapeDtypeStruct((_NPT, _CW), jnp.float32),
            jax.ShapeDtypeStruct((_NPT, _D), jnp.float32),
        ],
    )(x_p, wcat, g, b, bias)

    pad = _EP - _E
    src3 = jnp.concatenate(
        [edge_index[0], jnp.full((pad,), _N, jnp.int32)]).reshape(_NW, _CPW, _CHUNK)
    dst3 = jnp.concatenate(
        [edge_index[1], jnp.zeros((pad,), jnp.int32)]).reshape(_NW, _CPW, _CHUNK)
    zero = jnp.zeros((_RPT, _CW), jnp.float32)

    acc = _sc_scatter()(zaug, src3, dst3, zero)

    out = pl.pallas_call(
        _tc_post,
        grid=(_N // _BRP,),
        in_specs=[
            pl.BlockSpec((_NC, _BRP, _CW), lambda i: (0, i, 0)),
            pl.BlockSpec((_BRP, _D), lambda i: (i, 0)),
        ],
        out_specs=pl.BlockSpec((_BRP, _D), lambda i: (i, 0)),
        out_shape=jax.ShapeDtypeStruct((_N, _D), jnp.float32),
    )(acc, res)

    return (out, h)


# speedup vs baseline: 9.9552x; 1.6490x over previous
import functools

import jax
import jax.numpy as jnp
from jax import lax
from jax.experimental import pallas as pl
from jax.experimental.pallas import tpu as pltpu
from jax.experimental.pallas import tpu_sc as plsc

_N = 10000
_D = 128
_E = 320000

_NC = 2
_NS = 16
_NW = _NC * _NS
_CW = 144
_CHUNK = 64
_NBUF = 2
_RPT = _N // _NS
_EPW = _E // _NW
_CPW = _EPW // _CHUNK
_TAIL = _EPW - _CPW * _CHUNK
_BR = 1000
_BRP = 1000


def _tc_pre(x_ref, wcat_ref, g_ref, b_ref, bias_ref, zaug_ref, res_ref):
    xr = x_ref[...]
    mu = jnp.mean(xr, axis=1, keepdims=True)
    d = xr - mu
    var = jnp.mean(d * d, axis=1, keepdims=True)
    xn = d * lax.rsqrt(var + 1e-5) * g_ref[...] + b_ref[...]
    zz = lax.dot_general(xn, wcat_ref[...], (((1,), (0,)), ((), ())),
                         preferred_element_type=jnp.float32)
    res_ref[...] = zz[:, _D:] + xr + bias_ref[...]
    col = lax.broadcasted_iota(jnp.int32, (_BR, _CW - _D), 1)
    zaug_ref[:, :_D] = zz[:, :_D]
    zaug_ref[:, _D:] = jnp.where(col == 0, 1.0, 0.0).astype(jnp.float32)


def _tc_post(acc_ref, res_ref, out_ref):
    s = acc_ref[0] + acc_ref[1]
    agg = s[:, :_D]
    deg = s[:, _D:_D + 1]
    mean = agg / jnp.maximum(deg, 1.0)
    out_ref[...] = jnp.maximum(mean + res_ref[...], 0.0)


def _sc_body(zaug_hbm, src_hbm, dst_hbm, zero_hbm, out_hbm,
             src_v, dst_v, rows_v, acc_sh, sems):
    c = lax.axis_index("c")
    s = lax.axis_index("s")
    wid = c * _NS + s
    pltpu.sync_copy(zero_hbm, acc_sh.at[pl.ds(s * _RPT, _RPT)])
    pltpu.sync_copy(src_hbm.at[pl.ds(wid * _EPW, _EPW)], src_v)
    pltpu.sync_copy(dst_hbm.at[pl.ds(wid * _EPW, _EPW)], dst_v)
    plsc.subcore_barrier()

    for b in range(_NBUF):
        pltpu.async_copy(
            zaug_hbm.at[src_v.at[pl.ds(b * _CHUNK, _CHUNK)]],
            rows_v.at[b], sems.at[b])

    def body(t, carry):
        for b in range(_NBUF):
            j = t * _NBUF + b
            pltpu.make_async_copy(
                zaug_hbm.at[src_v.at[pl.ds(j * _CHUNK, _CHUNK)]],
                rows_v.at[b], sems.at[b]).wait()
            pltpu.sync_copy(rows_v.at[b],
                            acc_sh.at[dst_v.at[pl.ds(j * _CHUNK, _CHUNK)]],
                            add=True)

            @pl.when(j + _NBUF < _CPW)
            def _():
                pltpu.async_copy(
                    zaug_hbm.at[src_v.at[pl.ds((j + _NBUF) * _CHUNK, _CHUNK)]],
                    rows_v.at[b], sems.at[b])
        return carry

    lax.fori_loop(0, _CPW // _NBUF, body, 0)
    pltpu.sync_copy(
        zaug_hbm.at[src_v.at[pl.ds(_CPW * _CHUNK, _TAIL)]],
        rows_v.at[0, pl.ds(0, _TAIL)])
    pltpu.sync_copy(rows_v.at[0, pl.ds(0, _TAIL)],
                    acc_sh.at[dst_v.at[pl.ds(_CPW * _CHUNK, _TAIL)]],
                    add=True)
    plsc.subcore_barrier()
    pltpu.sync_copy(acc_sh.at[pl.ds(s * _RPT, _RPT)],
                    out_hbm.at[c, pl.ds(s * _RPT, _RPT)])


@functools.cache
def _sc_scatter():
    return pl.kernel(
        _sc_body,
        out_type=jax.ShapeDtypeStruct((_NC, _N, _CW), jnp.float32),
        mesh=plsc.VectorSubcoreMesh(core_axis_name="c", subcore_axis_name="s",
                                    num_cores=_NC, num_subcores=_NS),
        scratch_types=[
            pltpu.VMEM((_EPW,), jnp.int32),
            pltpu.VMEM((_EPW,), jnp.int32),
            pltpu.VMEM((_NBUF, _CHUNK, _CW), jnp.float32),
            pltpu.VMEM_SHARED((_N, _CW), jnp.float32),
            pltpu.SemaphoreType.DMA((_NBUF,)),
        ],
        compiler_params=pltpu.CompilerParams(use_tc_tiling_on_sc=False),
    )


def kernel(x, edge_index, edge_attr, h, batch, W_l, b_l, W_r, b_r, gamma, beta):
    wcat = jnp.concatenate([W_l.T, W_r.T], axis=1)
    bias = (b_l + b_r).reshape(1, _D)
    g = gamma.reshape(1, _D)
    b = beta.reshape(1, _D)

    zaug, res = pl.pallas_call(
        _tc_pre,
        grid=(_N // _BR,),
        in_specs=[
            pl.BlockSpec((_BR, _D), lambda i: (i, 0)),
            pl.BlockSpec((_D, 2 * _D), lambda i: (0, 0)),
            pl.BlockSpec((1, _D), lambda i: (0, 0)),
            pl.BlockSpec((1, _D), lambda i: (0, 0)),
            pl.BlockSpec((1, _D), lambda i: (0, 0)),
        ],
        out_specs=[
            pl.BlockSpec((_BR, _CW), lambda i: (i, 0)),
            pl.BlockSpec((_BR, _D), lambda i: (i, 0)),
        ],
        out_shape=[
            jax.ShapeDtypeStruct((_N, _CW), jnp.float32),
            jax.ShapeDtypeStruct((_N, _D), jnp.float32),
        ],
    )(x, wcat, g, b, bias)

    zero = jnp.zeros((_RPT, _CW), jnp.float32)
    acc = _sc_scatter()(zaug, edge_index[0], edge_index[1], zero)

    out = pl.pallas_call(
        _tc_post,
        grid=(_N // _BRP,),
        in_specs=[
            pl.BlockSpec((_NC, _BRP, _CW), lambda i: (0, i, 0)),
            pl.BlockSpec((_BRP, _D), lambda i: (i, 0)),
        ],
        out_specs=pl.BlockSpec((_BRP, _D), lambda i: (i, 0)),
        out_shape=jax.ShapeDtypeStruct((_N, _D), jnp.float32),
    )(acc, res)

    return (out, h)


# speedup vs baseline: 10.9763x; 1.1026x over previous
import functools

import jax
import jax.numpy as jnp
from jax import lax
from jax.experimental import pallas as pl
from jax.experimental.pallas import tpu as pltpu
from jax.experimental.pallas import tpu_sc as plsc

_N = 10000
_D = 128
_E = 320000

_NC = 2
_NS = 16
_NW = _NC * _NS
_DW = 16
_CHUNK = 64
_NBUF = 2
_RPT = _N // _NS
_EPW = _E // _NW
_CPW = _EPW // _CHUNK
_TAIL = _EPW - _CPW * _CHUNK
_BR = 1000
_BRP = 1000


def _tc_pre(x_ref, wcat_ref, g_ref, b_ref, bias_ref, z_ref, res_ref):
    xr = x_ref[...]
    mu = jnp.mean(xr, axis=1, keepdims=True)
    d = xr - mu
    var = jnp.mean(d * d, axis=1, keepdims=True)
    xn = d * lax.rsqrt(var + 1e-5) * g_ref[...] + b_ref[...]
    zz = lax.dot_general(xn, wcat_ref[...], (((1,), (0,)), ((), ())),
                         preferred_element_type=jnp.float32)
    res_ref[...] = zz[:, _D:] + xr + bias_ref[...]
    z_ref[...] = zz[:, :_D]


def _tc_post(acc_ref, deg_ref, res_ref, out_ref):
    agg = acc_ref[0] + acc_ref[1]
    deg = deg_ref[0, :, 0:1] + deg_ref[1, :, 0:1]
    mean = agg / jnp.maximum(deg, 1.0)
    out_ref[...] = jnp.maximum(mean + res_ref[...], 0.0)


def _sc_body(z_hbm, src_hbm, dst_hbm, zero_hbm, zerod_hbm, ones_hbm,
             out_hbm, outd_hbm,
             src_v, dst_v, rows_v, ones_v, acc_sh, deg_sh, sems):
    c = lax.axis_index("c")
    s = lax.axis_index("s")
    wid = c * _NS + s
    pltpu.sync_copy(zero_hbm, acc_sh.at[pl.ds(s * _RPT, _RPT)])
    pltpu.sync_copy(zerod_hbm, deg_sh.at[pl.ds(s * _RPT, _RPT)])
    pltpu.sync_copy(src_hbm.at[pl.ds(wid * _EPW, _EPW)], src_v)
    pltpu.sync_copy(dst_hbm.at[pl.ds(wid * _EPW, _EPW)], dst_v)
    pltpu.sync_copy(ones_hbm, ones_v)
    plsc.subcore_barrier()

    for b in range(_NBUF):
        pltpu.async_copy(
            z_hbm.at[src_v.at[pl.ds(b * _CHUNK, _CHUNK)]],
            rows_v.at[b], sems.at[b])

    def body(t, carry):
        for b in range(_NBUF):
            j = t * _NBUF + b
            pltpu.make_async_copy(
                z_hbm.at[src_v.at[pl.ds(j * _CHUNK, _CHUNK)]],
                rows_v.at[b], sems.at[b]).wait()
            pltpu.sync_copy(rows_v.at[b],
                            acc_sh.at[dst_v.at[pl.ds(j * _CHUNK, _CHUNK)]],
                            add=True)
            pltpu.sync_copy(ones_v,
                            deg_sh.at[dst_v.at[pl.ds(j * _CHUNK, _CHUNK)]],
                            add=True)

            @pl.when(j + _NBUF < _CPW)
            def _():
                pltpu.async_copy(
                    z_hbm.at[src_v.at[pl.ds((j + _NBUF) * _CHUNK, _CHUNK)]],
                    rows_v.at[b], sems.at[b])
        return carry

    lax.fori_loop(0, _CPW // _NBUF, body, 0)
    pltpu.sync_copy(
        z_hbm.at[src_v.at[pl.ds(_CPW * _CHUNK, _TAIL)]],
        rows_v.at[0, pl.ds(0, _TAIL)])
    pltpu.sync_copy(rows_v.at[0, pl.ds(0, _TAIL)],
                    acc_sh.at[dst_v.at[pl.ds(_CPW * _CHUNK, _TAIL)]],
                    add=True)
    pltpu.sync_copy(ones_v.at[pl.ds(0, _TAIL)],
                    deg_sh.at[dst_v.at[pl.ds(_CPW * _CHUNK, _TAIL)]],
                    add=True)
    plsc.subcore_barrier()
    pltpu.sync_copy(acc_sh.at[pl.ds(s * _RPT, _RPT)],
                    out_hbm.at[c, pl.ds(s * _RPT, _RPT)])
    pltpu.sync_copy(deg_sh.at[pl.ds(s * _RPT, _RPT)],
                    outd_hbm.at[c, pl.ds(s * _RPT, _RPT)])


@functools.cache
def _sc_scatter():
    return pl.kernel(
        _sc_body,
        out_type=(
            jax.ShapeDtypeStruct((_NC, _N, _D), jnp.float32),
            jax.ShapeDtypeStruct((_NC, _N, _DW), jnp.float32),
        ),
        mesh=plsc.VectorSubcoreMesh(core_axis_name="c", subcore_axis_name="s",
                                    num_cores=_NC, num_subcores=_NS),
        scratch_types=[
            pltpu.VMEM((_EPW,), jnp.int32),
            pltpu.VMEM((_EPW,), jnp.int32),
            pltpu.VMEM((_NBUF, _CHUNK, _D), jnp.float32),
            pltpu.VMEM((_CHUNK, _DW), jnp.float32),
            pltpu.VMEM_SHARED((_N, _D), jnp.float32),
            pltpu.VMEM_SHARED((_N, _DW), jnp.float32),
            pltpu.SemaphoreType.DMA((_NBUF,)),
        ],
        compiler_params=pltpu.CompilerParams(use_tc_tiling_on_sc=False),
    )


def kernel(x, edge_index, edge_attr, h, batch, W_l, b_l, W_r, b_r, gamma, beta):
    wcat = jnp.concatenate([W_l.T, W_r.T], axis=1)
    bias = (b_l + b_r).reshape(1, _D)
    g = gamma.reshape(1, _D)
    b = beta.reshape(1, _D)

    z, res = pl.pallas_call(
        _tc_pre,
        grid=(_N // _BR,),
        in_specs=[
            pl.BlockSpec((_BR, _D), lambda i: (i, 0)),
            pl.BlockSpec((_D, 2 * _D), lambda i: (0, 0)),
            pl.BlockSpec((1, _D), lambda i: (0, 0)),
            pl.BlockSpec((1, _D), lambda i: (0, 0)),
            pl.BlockSpec((1, _D), lambda i: (0, 0)),
        ],
        out_specs=[
            pl.BlockSpec((_BR, _D), lambda i: (i, 0)),
            pl.BlockSpec((_BR, _D), lambda i: (i, 0)),
        ],
        out_shape=[
            jax.ShapeDtypeStruct((_N, _D), jnp.float32),
            jax.ShapeDtypeStruct((_N, _D), jnp.float32),
        ],
    )(x, wcat, g, b, bias)

    zero = jnp.zeros((_RPT, _D), jnp.float32)
    zerod = jnp.zeros((_RPT, _DW), jnp.float32)
    ones = jnp.zeros((_CHUNK, _DW), jnp.float32).at[:, 0].set(1.0)
    acc, dega = _sc_scatter()(z, edge_index[0], edge_index[1],
                              zero, zerod, ones)

    out = pl.pallas_call(
        _tc_post,
        grid=(_N // _BRP,),
        in_specs=[
            pl.BlockSpec((_NC, _BRP, _D), lambda i: (0, i, 0)),
            pl.BlockSpec((_NC, _BRP, _DW), lambda i: (0, i, 0)),
            pl.BlockSpec((_BRP, _D), lambda i: (i, 0)),
        ],
        out_specs=pl.BlockSpec((_BRP, _D), lambda i: (i, 0)),
        out_shape=jax.ShapeDtypeStruct((_N, _D), jnp.float32),
    )(acc, dega, res)

    return (out, h)
